# baseline (speedup 1.0000x reference)
import functools

import jax
import jax.numpy as jnp
from jax import lax
from jax.experimental import pallas as pl
from jax.experimental.pallas import tpu as pltpu
from jax.experimental.pallas import tpu_sc as plsc

DH = 128
DEPTH = 3
L = 16
VR = DH // L
T = 16
NB = 2
_PREC = lax.Precision.HIGHEST


def _mm(a, b):
    ah = a.astype(jnp.bfloat16)
    al = (a - ah.astype(jnp.float32)).astype(jnp.bfloat16)
    bh = b.astype(jnp.bfloat16)
    bl = (b - bh.astype(jnp.float32)).astype(jnp.bfloat16)
    dn = (((1,), (0,)), ((), ()))
    f32 = jnp.float32
    hi = lax.dot_general(ah, bh, dn, preferred_element_type=f32)
    m1 = lax.dot_general(ah, bl, dn, preferred_element_type=f32)
    m2 = lax.dot_general(al, bh, dn, preferred_element_type=f32)
    return hi + m1 + m2


def _pack16(a):
    half = DH // 2
    lo = lax.bitcast_convert_type(a[:, :half].astype(jnp.bfloat16), jnp.uint16)
    hi = lax.bitcast_convert_type(a[:, half:].astype(jnp.bfloat16), jnp.uint16)
    return lo.astype(jnp.int32) | (hi.astype(jnp.int32) << 16)


def _init_body(x_ref, h_ref, wzx, whx, wr, ur, bur, table_ref, xz_ref, xh_ref, r1_ref):
    xb = x_ref[...]
    hb = h_ref[...]
    table_ref[:, :DH // 2] = _pack16(hb)
    table_ref[:, DH // 2:] = _pack16(jnp.exp(-_mm(hb, ur[...])))
    xz_ref[...] = _mm(xb, wzx[...])
    xh_ref[...] = _mm(xb, whx[...])
    r1_ref[...] = jnp.exp(-(_mm(xb, wr[...]) + bur[...]))


def _update_body(br, last, xz_ref, xh_ref, sums_ref, wzh, whh, bz, bh, ur, out_ref):
    sh = sums_ref[:, :DH]
    sg = sums_ref[:, DH:]
    z = jax.nn.sigmoid(xz_ref[...] + _mm(sh, wzh[...]) + bz[...])
    pre = jnp.tanh(xh_ref[...] + _mm(sg, whh[...]) + bh[...])
    hn = (1.0 - z) * sh + z * pre
    rowid = lax.broadcasted_iota(jnp.int32, hn.shape, 0) + pl.program_id(0) * br
    hn = jnp.where(rowid == 0, 0.0, hn)
    if last:
        out_ref[...] = hn
    else:
        out_ref[:, :DH // 2] = _pack16(hn)
        out_ref[:, DH // 2:] = _pack16(jnp.exp(-_mm(hn, ur[...])))


def _sc_gather_sums(npad, pw0, pw1, nc, k_deg, table, idx_flat, r1):
    e_blk = T * k_deg
    pwmax = max(pw0, pw1)
    mesh = plsc.VectorSubcoreMesh(core_axis_name="c", subcore_axis_name="s")

    @functools.partial(
        pl.kernel,
        mesh=mesh,
        out_type=jax.ShapeDtypeStruct((npad, 2 * DH), jnp.float32),
        scratch_types=[pltpu.VMEM((pwmax * k_deg,), jnp.int32)]
        + [pltpu.VMEM((e_blk, DH), jnp.int32)] * NB
        + [pltpu.VMEM((T, DH), jnp.float32)] * NB
        + [pltpu.VMEM((T, 2 * DH), jnp.float32)] * NB
        + [pltpu.SemaphoreType.DMA] * (3 * NB),
    )
    def sc_kernel(table_hbm, idx_hbm, r1_hbm, out_hbm, idx_v, *bufs):
        rows = bufs[0:NB]
        r1v = bufs[NB:2 * NB]
        sums = bufs[2 * NB:3 * NB]
        gsem = bufs[3 * NB:4 * NB]
        rsem = bufs[4 * NB:5 * NB]
        osem = bufs[5 * NB:6 * NB]
        c = lax.axis_index("c")
        s_id = lax.axis_index("s")
        base = jnp.where(c == 0, s_id * pw0, 16 * pw0 + s_id * pw1)
        nblocks = jnp.where(c == 0, pw0 // T, pw1 // T)
        pltpu.sync_copy(idx_hbm.at[pl.ds(base * k_deg, pwmax * k_deg)], idx_v)

        def g_copy(j, b):
            return pltpu.make_async_copy(
                table_hbm.at[idx_v.at[pl.ds(j * e_blk, e_blk)]], rows[b], gsem[b])

        def r_copy(j, b):
            return pltpu.make_async_copy(
                r1_hbm.at[pl.ds(base + j * T, T)], r1v[b], rsem[b])

        def o_copy(j, b):
            return pltpu.make_async_copy(
                sums[b], out_hbm.at[pl.ds(base + j * T, T)], osem[b])

        def compute(b):
            rb = rows[b]
            sb = sums[b]
            nw = DH // (2 * L)
            f32 = jnp.float32

            def unpair(w):
                lo = lax.bitcast_convert_type(w << 16, f32)
                hi = lax.bitcast_convert_type(w & jnp.int32(-65536), f32)
                return lo, hi

            @plsc.parallel_loop(0, T, 1)
            def node(t):
                e1vs = [r1v[b][t, pl.ds(L * v, L)] for v in range(VR)]
                acc_h = [None] * VR
                acc_g = [None] * VR
                for kk in range(k_deg):
                    e = t * k_deg + kk
                    for u in range(nw):
                        hlo, hhi = unpair(rb[e, pl.ds(L * u, L)])
                        elo, ehi = unpair(rb[e, pl.ds(DH // 2 + L * u, L)])
                        for v, hv, ev in ((u, hlo, elo), (u + nw, hhi, ehi)):
                            s = 1.0 / (1.0 + e1vs[v] * ev)
                            g = s * hv
                            if kk == 0:
                                acc_h[v] = hv
                                acc_g[v] = g
                            else:
                                acc_h[v] = acc_h[v] + hv
                                acc_g[v] = acc_g[v] + g
                for v in range(VR):
                    sb[t, pl.ds(L * v, L)] = acc_h[v]
                    sb[t, pl.ds(DH + L * v, L)] = acc_g[v]

        def step(j, b, first):
            g_copy(j, b).wait()
            r_copy(j, b).wait()
            if not first:
                o_copy(j, b).wait()
            compute(b)
            o_copy(j, b).start()
            jn = jnp.minimum(j + NB, nblocks - 1)
            g_copy(jn, b).start()
            r_copy(jn, b).start()

        for b in range(NB):
            g_copy(b, b).start()
            r_copy(b, b).start()
        for b in range(NB):
            step(b, b, True)

        def outer(jj, c):
            for b in range(NB):
                step(jj * NB + b, b, False)
            return c

        lax.fori_loop(1, nblocks // NB, outer, 0, unroll=False)
        for b in range(NB):
            g_copy(0, b).wait()
            r_copy(0, b).wait()
            o_copy(0, b).wait()

    return sc_kernel(table, idx_flat, r1)


def kernel(h, x, mess_graph, W_z, b_z, W_r, U_r, b_ur, W_h, b_h):
    n, dh = h.shape
    di = x.shape[1]
    k_deg = mess_graph.shape[1]

    info = plsc.get_sparse_core_info()
    nc, ns = info.num_cores, info.num_subcores
    nw = nc * ns
    per_w = -(-n // nw)
    per_w = -(-per_w // (NB * T)) * (NB * T)
    skew = 192
    pw0, pw1 = per_w - skew, per_w + skew
    npad = ns * (pw0 + pw1)

    idx_flat = jnp.zeros(((npad + max(pw0, pw1)) * k_deg,), jnp.int32).at[: n * k_deg].set(
        mess_graph.astype(jnp.int32).reshape(-1))
    wzx = W_z[:, :di].T
    wzh = W_z[:, di:].T
    whx = W_h[:, :di].T
    whh = W_h[:, di:].T
    wr = W_r.T
    ur = U_r.T
    bz = b_z.reshape(1, dh)
    bh = b_h.reshape(1, dh)
    bur = b_ur.reshape(1, dh)

    br = 512
    grid = (npad // br,)
    row_spec = lambda w: pl.BlockSpec((br, w), lambda i: (i, 0))
    wt_spec = pl.BlockSpec((dh, dh), lambda i: (0, 0))
    b_spec = pl.BlockSpec((1, dh), lambda i: (0, 0))

    table, xz, xh, r1 = pl.pallas_call(
        _init_body,
        grid=grid,
        in_specs=[row_spec(di), row_spec(dh), wt_spec, wt_spec, wt_spec, wt_spec, b_spec],
        out_specs=[row_spec(dh), row_spec(dh), row_spec(dh), row_spec(dh)],
        out_shape=[
            jax.ShapeDtypeStruct((npad, dh), jnp.int32),
            jax.ShapeDtypeStruct((npad, dh), jnp.float32),
            jax.ShapeDtypeStruct((npad, dh), jnp.float32),
            jax.ShapeDtypeStruct((npad, dh), jnp.float32),
        ],
    )(x, h, wzx, whx, wr, ur, bur)

    for depth in range(DEPTH):
        sums = _sc_gather_sums(npad, pw0, pw1, nc, k_deg, table, idx_flat, r1)
        last = depth == DEPTH - 1
        out_w = dh
        out_rows = n if last else npad
        table = pl.pallas_call(
            functools.partial(_update_body, br, last),
            grid=grid,
            in_specs=[row_spec(dh), row_spec(dh), row_spec(2 * dh),
                      wt_spec, wt_spec, b_spec, b_spec, wt_spec],
            out_specs=row_spec(out_w),
            out_shape=jax.ShapeDtypeStruct(
                (out_rows, out_w), jnp.float32 if last else jnp.int32),
        )(xz, xh, sums, wzh, whh, bz, bh, ur)

    return table


# speedup vs baseline: 1.0352x; 1.0352x over previous
import functools

import jax
import jax.numpy as jnp
from jax import lax
from jax.experimental import pallas as pl
from jax.experimental.pallas import tpu as pltpu
from jax.experimental.pallas import tpu_sc as plsc

DH = 128
DEPTH = 3
L = 16
VR = DH // L
T = 16
NB = 2
_PREC = lax.Precision.HIGHEST


def _mm(a, b):
    ah = a.astype(jnp.bfloat16)
    al = (a - ah.astype(jnp.float32)).astype(jnp.bfloat16)
    bh = b.astype(jnp.bfloat16)
    bl = (b - bh.astype(jnp.float32)).astype(jnp.bfloat16)
    dn = (((1,), (0,)), ((), ()))
    f32 = jnp.float32
    hi = lax.dot_general(ah, bh, dn, preferred_element_type=f32)
    m1 = lax.dot_general(ah, bl, dn, preferred_element_type=f32)
    m2 = lax.dot_general(al, bh, dn, preferred_element_type=f32)
    return hi + m1 + m2


def _pack16(a):
    half = DH // 2
    lo = lax.bitcast_convert_type(a[:, :half].astype(jnp.bfloat16), jnp.uint16)
    hi = lax.bitcast_convert_type(a[:, half:].astype(jnp.bfloat16), jnp.uint16)
    return lo.astype(jnp.int32) | (hi.astype(jnp.int32) << 16)


def _init_body(x_ref, h_ref, wzx, whx, wr, ur, bur, table_ref, xz_ref, xh_ref, r1_ref):
    xb = x_ref[...]
    hb = h_ref[...]
    table_ref[:, :DH // 2] = _pack16(hb)
    table_ref[:, DH // 2:] = _pack16(jnp.exp(-_mm(hb, ur[...])))
    xz_ref[...] = _mm(xb, wzx[...])
    xh_ref[...] = _mm(xb, whx[...])
    r1_ref[...] = jnp.exp(-(_mm(xb, wr[...]) + bur[...]))


def _update_body(br, last, xz_ref, xh_ref, sums_ref, wzh, whh, bz, bh, ur, out_ref):
    sh = sums_ref[:, :DH]
    sg = sums_ref[:, DH:]
    z = jax.nn.sigmoid(xz_ref[...] + _mm(sh, wzh[...]) + bz[...])
    pre = jnp.tanh(xh_ref[...] + _mm(sg, whh[...]) + bh[...])
    hn = (1.0 - z) * sh + z * pre
    rowid = lax.broadcasted_iota(jnp.int32, hn.shape, 0) + pl.program_id(0) * br
    hn = jnp.where(rowid == 0, 0.0, hn)
    if last:
        out_ref[...] = hn
    else:
        out_ref[:, :DH // 2] = _pack16(hn)
        out_ref[:, DH // 2:] = _pack16(jnp.exp(-_mm(hn, ur[...])))


def _sc_gather_sums(npad, pw0, pw1, nc, k_deg, table, idx_flat, r1):
    e_blk = T * k_deg
    pwmax = max(pw0, pw1)
    mesh = plsc.VectorSubcoreMesh(core_axis_name="c", subcore_axis_name="s")

    @functools.partial(
        pl.kernel,
        mesh=mesh,
        out_type=jax.ShapeDtypeStruct((npad, 2 * DH), jnp.float32),
        scratch_types=[pltpu.VMEM((pwmax * k_deg,), jnp.int32)]
        + [pltpu.VMEM((e_blk, DH), jnp.int32)] * NB
        + [pltpu.VMEM((T, DH), jnp.float32)] * NB
        + [pltpu.VMEM((T, 2 * DH), jnp.float32)] * NB
        + [pltpu.SemaphoreType.DMA] * (3 * NB),
    )
    def sc_kernel(table_hbm, idx_hbm, r1_hbm, out_hbm, idx_v, *bufs):
        rows = bufs[0:NB]
        r1v = bufs[NB:2 * NB]
        sums = bufs[2 * NB:3 * NB]
        gsem = bufs[3 * NB:4 * NB]
        rsem = bufs[4 * NB:5 * NB]
        osem = bufs[5 * NB:6 * NB]
        c = lax.axis_index("c")
        s_id = lax.axis_index("s")
        base = jnp.where(c == 0, s_id * pw0, 16 * pw0 + s_id * pw1)
        nblocks = jnp.where(c == 0, pw0 // T, pw1 // T)
        pltpu.sync_copy(idx_hbm.at[pl.ds(base * k_deg, pwmax * k_deg)], idx_v)

        def g_copy(j, b):
            return pltpu.make_async_copy(
                table_hbm.at[idx_v.at[pl.ds(j * e_blk, e_blk)]], rows[b], gsem[b])

        def r_copy(j, b):
            return pltpu.make_async_copy(
                r1_hbm.at[pl.ds(base + j * T, T)], r1v[b], rsem[b])

        def o_copy(j, b):
            return pltpu.make_async_copy(
                sums[b], out_hbm.at[pl.ds(base + j * T, T)], osem[b])

        def compute(b):
            rb = rows[b]
            sb = sums[b]
            nw = DH // (2 * L)
            f32 = jnp.float32

            def unpair(w):
                lo = lax.bitcast_convert_type(w << 16, f32)
                hi = lax.bitcast_convert_type(w & jnp.int32(-65536), f32)
                return lo, hi

            @plsc.parallel_loop(0, T, 1)
            def node(t):
                e1vs = [r1v[b][t, pl.ds(L * v, L)] for v in range(VR)]
                acc_h = [None] * VR
                acc_g = [None] * VR
                for kk in range(k_deg):
                    e = t * k_deg + kk
                    for u in range(nw):
                        hlo, hhi = unpair(rb[e, pl.ds(L * u, L)])
                        elo, ehi = unpair(rb[e, pl.ds(DH // 2 + L * u, L)])
                        for v, hv, ev in ((u, hlo, elo), (u + nw, hhi, ehi)):
                            s = 1.0 / (1.0 + e1vs[v] * ev)
                            g = s * hv
                            if kk == 0:
                                acc_h[v] = hv
                                acc_g[v] = g
                            else:
                                acc_h[v] = acc_h[v] + hv
                                acc_g[v] = acc_g[v] + g
                for v in range(VR):
                    sb[t, pl.ds(L * v, L)] = acc_h[v]
                    sb[t, pl.ds(DH + L * v, L)] = acc_g[v]

        def step(j, b, first):
            g_copy(j, b).wait()
            r_copy(j, b).wait()
            if not first:
                o_copy(j, b).wait()
            compute(b)
            o_copy(j, b).start()
            jn = jnp.minimum(j + NB, nblocks - 1)
            g_copy(jn, b).start()
            r_copy(jn, b).start()

        for b in range(NB):
            g_copy(b, b).start()
            r_copy(b, b).start()
        for b in range(NB):
            step(b, b, True)

        def outer(jj, c):
            for b in range(NB):
                step(jj * NB + b, b, False)
            return c

        lax.fori_loop(1, nblocks // NB, outer, 0, unroll=False)
        for b in range(NB):
            g_copy(0, b).wait()
            r_copy(0, b).wait()
            o_copy(0, b).wait()

    return sc_kernel(table, idx_flat, r1)


def kernel(h, x, mess_graph, W_z, b_z, W_r, U_r, b_ur, W_h, b_h):
    n, dh = h.shape
    di = x.shape[1]
    k_deg = mess_graph.shape[1]

    info = plsc.get_sparse_core_info()
    nc, ns = info.num_cores, info.num_subcores
    nw = nc * ns
    per_w = -(-n // nw)
    per_w = -(-per_w // (NB * T)) * (NB * T)
    skew = 192
    pw0, pw1 = per_w + skew, per_w - skew
    npad = ns * (pw0 + pw1)

    idx_flat = jnp.zeros(((npad + max(pw0, pw1)) * k_deg,), jnp.int32).at[: n * k_deg].set(
        mess_graph.astype(jnp.int32).reshape(-1))
    wzx = W_z[:, :di].T
    wzh = W_z[:, di:].T
    whx = W_h[:, :di].T
    whh = W_h[:, di:].T
    wr = W_r.T
    ur = U_r.T
    bz = b_z.reshape(1, dh)
    bh = b_h.reshape(1, dh)
    bur = b_ur.reshape(1, dh)

    br = 512
    grid = (npad // br,)
    row_spec = lambda w: pl.BlockSpec((br, w), lambda i: (i, 0))
    wt_spec = pl.BlockSpec((dh, dh), lambda i: (0, 0))
    b_spec = pl.BlockSpec((1, dh), lambda i: (0, 0))

    table, xz, xh, r1 = pl.pallas_call(
        _init_body,
        grid=grid,
        in_specs=[row_spec(di), row_spec(dh), wt_spec, wt_spec, wt_spec, wt_spec, b_spec],
        out_specs=[row_spec(dh), row_spec(dh), row_spec(dh), row_spec(dh)],
        out_shape=[
            jax.ShapeDtypeStruct((npad, dh), jnp.int32),
            jax.ShapeDtypeStruct((npad, dh), jnp.float32),
            jax.ShapeDtypeStruct((npad, dh), jnp.float32),
            jax.ShapeDtypeStruct((npad, dh), jnp.float32),
        ],
    )(x, h, wzx, whx, wr, ur, bur)

    for depth in range(DEPTH):
        sums = _sc_gather_sums(npad, pw0, pw1, nc, k_deg, table, idx_flat, r1)
        last = depth == DEPTH - 1
        out_w = dh
        out_rows = n if last else npad
        table = pl.pallas_call(
            functools.partial(_update_body, br, last),
            grid=grid,
            in_specs=[row_spec(dh), row_spec(dh), row_spec(2 * dh),
                      wt_spec, wt_spec, b_spec, b_spec, wt_spec],
            out_specs=row_spec(out_w),
            out_shape=jax.ShapeDtypeStruct(
                (out_rows, out_w), jnp.float32 if last else jnp.int32),
        )(xz, xh, sums, wzh, whh, bz, bh, ur)

    return table
